# Initial kernel scaffold; baseline (speedup 1.0000x reference)
import jax
import jax.numpy as jnp
from jax.experimental import pallas as pl


def kernel(obs_emb, adj, observed_mask, observed_tp, tp_emb_tensor, lengths, avg_interval, var_prior_emb_tensor, rarity_W, Wg1, bg1, Wg2, bg2, Wu, bu, Wr, br, Wc, bc):
    raise NotImplementedError("write your pallas kernel here")


# speedup vs baseline: 1.4648x; 1.4648x over previous
import functools

import jax
import jax.numpy as jnp
from jax.experimental import pallas as pl
from jax.experimental.pallas import tpu as pltpu

B, T, N, D = 128, 24, 100, 64
RARITY_ALPHA = 0.5
BB = 64
NB = B // BB


def _step_kernel(obs_ref, maskt_ref, maskfull_ref, avgt_ref, len_ref,
                 rarw_ref, vp_ref, wg1_ref, bg1_ref, wg2_ref, bg2_ref,
                 wr_ref, wsr_ref, br_ref,
                 wu_ref, wsu_ref, bu_ref,
                 wc_ref, wsc_ref, bc_ref,
                 out_ref, h_ref, adj_ref, vt_ref):
    t = pl.program_id(1)

    @pl.when(t == 0)
    def _init():
        pg = jnp.maximum(
            jax.lax.dot(vp_ref[...], wg1_ref[...],
                        preferred_element_type=jnp.float32) + bg1_ref[...],
            0.0)
        pg = jax.lax.dot(pg, wg2_ref[...],
                         preferred_element_type=jnp.float32) + bg2_ref[...]
        nrm = jnp.sqrt(jnp.sum(pg * pg, axis=-1, keepdims=True))
        vn = pg / jnp.maximum(nrm, 1e-12)
        logits = jax.lax.dot_general(
            vn, vn, (((1,), (1,)), ((), ())),
            preferred_element_type=jnp.float32)
        mx = jnp.max(logits, axis=-1, keepdims=True)
        e = jnp.exp(logits - mx)
        adj_ref[...] = e / jnp.sum(e, axis=-1, keepdims=True)
        vt_ref[...] = jnp.sum(maskfull_ref[...], axis=0)
        h_ref[...] = jnp.zeros_like(h_ref)
        out_ref[...] = jnp.zeros_like(out_ref)

    mask = maskt_ref[0]
    avg = avgt_ref[0]
    vt = vt_ref[...]
    h = h_ref[...]
    cur_obs = obs_ref[:, 0]

    rs = RARITY_ALPHA * jnp.tanh(avg / (vt + 1.0))
    diff = jnp.abs(rs[:, :, None] - rs[:, None, :])
    row_i = jax.lax.broadcasted_iota(jnp.int32, (N, N), 0)
    col_i = jax.lax.broadcasted_iota(jnp.int32, (N, N), 1)
    eye = (row_i == col_i).astype(jnp.float32)
    off = (adj_ref[...] * (1.0 - eye))[None]
    madj = mask[:, :, None] * mask[:, None, :]
    cur_adj = off * (1.0 - rarw_ref[...][None] * diff) * madj + eye[None]

    xh = jnp.concatenate([cur_obs, h], axis=-1)
    comb = jax.lax.dot_general(
        cur_adj, xh, (((2,), (1,)), ((0,), (0,))),
        preferred_element_type=jnp.float32)
    comb_s = jnp.sum(cur_adj * rs[:, None, :], axis=2)

    def gate(x, w_ref, ws_ref, b_ref, s):
        pre = jax.lax.dot_general(
            x, w_ref[...], (((2,), (1,)), ((1,), (0,))),
            preferred_element_type=jnp.float32)
        pre = jnp.transpose(pre, (1, 0, 2))
        return pre + s[:, :, None] * ws_ref[...][None] + b_ref[...][None]

    r = jax.nn.sigmoid(gate(comb, wr_ref, wsr_ref, br_ref, comb_s))
    u = jax.nn.sigmoid(gate(comb, wu_ref, wsu_ref, bu_ref, comb_s))

    m = mask[:, :, None]
    h_reset = h * (1.0 + m * (r - 1.0))
    xh_new = jnp.concatenate([cur_obs, h_reset], axis=-1)
    cand = jnp.tanh(gate(xh_new, wc_ref, wsc_ref, bc_ref, rs))
    mu = m * u
    h_next = h_reset * (1.0 - mu) + mu * cand
    h_ref[...] = h_next

    end = (len_ref[...] - 1 == t)
    out_ref[...] = jnp.where(end[:, :, None], h_next, out_ref[...])


@functools.partial(jax.jit, static_argnames=())
def _run(obs_emb, maskt, avgt, lengths, rarity_W, vp, Wg1, bg1, Wg2, bg2,
         Wr2, wsr, br2, Wu2, wsu, bu2, Wc2, wsc, bc2):
    grid = (NB, T)
    specs = [
        pl.BlockSpec((BB, 1, N, D), lambda bc, t: (bc, t, 0, 0)),
        pl.BlockSpec((1, BB, N), lambda bc, t: (t, bc, 0)),
        pl.BlockSpec((T, BB, N), lambda bc, t: (0, bc, 0)),
        pl.BlockSpec((1, BB, N), lambda bc, t: (t, bc, 0)),
        pl.BlockSpec((BB, 1), lambda bc, t: (bc, 0)),
        pl.BlockSpec((N, N), lambda bc, t: (0, 0)),
        pl.BlockSpec(vp.shape, lambda bc, t: (0, 0)),
        pl.BlockSpec(Wg1.shape, lambda bc, t: (0, 0)),
        pl.BlockSpec(bg1.shape, lambda bc, t: (0, 0)),
        pl.BlockSpec(Wg2.shape, lambda bc, t: (0, 0)),
        pl.BlockSpec(bg2.shape, lambda bc, t: (0, 0)),
    ]
    for w in (Wr2, wsr, br2, Wu2, wsu, bu2, Wc2, wsc, bc2):
        specs.append(
            pl.BlockSpec(w.shape, lambda bc, t, nd=w.ndim: (0,) * nd))
    return pl.pallas_call(
        _step_kernel,
        grid=grid,
        in_specs=specs,
        out_specs=pl.BlockSpec((BB, N, D), lambda bc, t: (bc, 0, 0)),
        out_shape=jax.ShapeDtypeStruct((B, N, D), jnp.float32),
        scratch_shapes=[
            pltpu.VMEM((BB, N, D), jnp.float32),
            pltpu.VMEM((N, N), jnp.float32),
            pltpu.VMEM((BB, N), jnp.float32),
        ],
        compiler_params=pltpu.CompilerParams(
            dimension_semantics=("arbitrary", "arbitrary"),
        ),
    )(obs_emb, maskt, maskt, avgt, lengths, rarity_W, vp, Wg1, bg1, Wg2,
      bg2, Wr2, wsr, br2, Wu2, wsu, bu2, Wc2, wsc, bc2)


def kernel(obs_emb, adj, observed_mask, observed_tp, tp_emb_tensor, lengths,
           avg_interval, var_prior_emb_tensor, rarity_W, Wg1, bg1, Wg2, bg2,
           Wu, bu, Wr, br, Wc, bc):
    del adj, observed_tp, tp_emb_tensor
    maskt = observed_mask.astype(jnp.float32).transpose(1, 0, 2)
    avgt = avg_interval.transpose(1, 0, 2)

    def split(w):
        return (jnp.concatenate([w[:, :D, :], w[:, D + 1:, :]], axis=1),
                w[:, D, :])

    Wr2, wsr = split(Wr)
    Wu2, wsu = split(Wu)
    Wc2, wsc = split(Wc)
    return _run(obs_emb, maskt, avgt, lengths, rarity_W,
                var_prior_emb_tensor, Wg1, bg1.reshape(1, -1), Wg2,
                bg2.reshape(1, -1), Wr2, wsr, br, Wu2, wsu, bu,
                Wc2, wsc, bc)


# speedup vs baseline: 1.7174x; 1.1724x over previous
import functools

import jax
import jax.numpy as jnp
from jax.experimental import pallas as pl
from jax.experimental.pallas import tpu as pltpu

B, T, N, D = 128, 24, 100, 64
RARITY_ALPHA = 0.5
BB = 128
NB = B // BB


def _step_kernel(obs_ref, maskt_ref, maskfull_ref, avgt_ref, len_ref,
                 rarw_ref, vp_ref, wg1_ref, bg1_ref, wg2_ref, bg2_ref,
                 wru_ref, bru_ref, wc_ref, bc_ref,
                 out_ref, h_ref, off_ref, roff_ref, vt_ref):
    t = pl.program_id(1)

    @pl.when(t == 0)
    def _init():
        pg = jnp.maximum(
            jax.lax.dot(vp_ref[...], wg1_ref[...],
                        preferred_element_type=jnp.float32) + bg1_ref[...],
            0.0)
        pg = jax.lax.dot(pg, wg2_ref[...],
                         preferred_element_type=jnp.float32) + bg2_ref[...]
        nrm = jnp.sqrt(jnp.sum(pg * pg, axis=-1, keepdims=True))
        vn = pg / jnp.maximum(nrm, 1e-12)
        logits = jax.lax.dot_general(
            vn, vn, (((1,), (1,)), ((), ())),
            preferred_element_type=jnp.float32)
        mx = jnp.max(logits, axis=-1, keepdims=True)
        e = jnp.exp(logits - mx)
        adj = e / jnp.sum(e, axis=-1, keepdims=True)
        row_i = jax.lax.broadcasted_iota(jnp.int32, (N, N), 0)
        col_i = jax.lax.broadcasted_iota(jnp.int32, (N, N), 1)
        noteye = (row_i != col_i).astype(jnp.float32)
        off = adj * noteye
        off_ref[...] = off.astype(jnp.bfloat16)
        roff_ref[...] = (off * rarw_ref[...]).astype(jnp.bfloat16)
        vt_ref[...] = jnp.sum(maskfull_ref[...], axis=0)
        h_ref[...] = jnp.zeros_like(h_ref)
        out_ref[...] = jnp.zeros_like(out_ref)

    mask = maskt_ref[0].astype(jnp.bfloat16)
    avg = avgt_ref[0]
    vt = vt_ref[...]
    h = h_ref[...]
    cur_obs = obs_ref[:, 0].astype(jnp.bfloat16)

    rs = (RARITY_ALPHA * jnp.tanh(avg / (vt + 1.0))).astype(jnp.bfloat16)
    diff = jnp.abs(rs[:, :, None] - rs[:, None, :])
    madj = mask[:, :, None] * mask[:, None, :]
    row_i = jax.lax.broadcasted_iota(jnp.int32, (N, N), 0)
    col_i = jax.lax.broadcasted_iota(jnp.int32, (N, N), 1)
    eyeb = (row_i == col_i).astype(jnp.bfloat16)
    cur_adj = (off_ref[...][None] - roff_ref[...][None] * diff) * madj \
        + eyeb[None]

    xh = jnp.concatenate([cur_obs, h, rs[:, :, None]], axis=-1)
    comb = jax.lax.dot_general(
        cur_adj, xh, (((2,), (1,)), ((0,), (0,))),
        preferred_element_type=jnp.float32
    ).astype(jnp.bfloat16)

    def gate(x, w_ref, b_ref):
        pre = jax.lax.dot_general(
            x, w_ref[...], (((2,), (1,)), ((1,), (0,))),
            preferred_element_type=jnp.float32)
        pre = jnp.transpose(pre, (1, 0, 2))
        return pre + b_ref[...][None]

    ru = jax.nn.sigmoid(gate(comb, wru_ref, bru_ref)).astype(jnp.bfloat16)
    r = ru[:, :, :D]
    u = ru[:, :, D:]

    m = mask[:, :, None]
    h_reset = h * (1.0 + m * (r - 1.0))
    xh_new = jnp.concatenate([cur_obs, h_reset, rs[:, :, None]], axis=-1)
    cand = jnp.tanh(gate(xh_new, wc_ref, bc_ref)).astype(jnp.bfloat16)
    mu = m * u
    h_next = h_reset * (1.0 - mu) + mu * cand
    h_ref[...] = h_next

    end = (len_ref[...] - 1 == t)
    out_ref[...] = jnp.where(end[:, :, None],
                             h_next.astype(jnp.float32), out_ref[...])


@jax.jit
def _run(obs_emb, maskt, avgt, lengths, rarity_W, vp, Wg1, bg1, Wg2, bg2,
         Wru2, bru2, Wc2, bc2):
    grid = (NB, T)
    specs = [
        pl.BlockSpec((BB, 1, N, D), lambda bc, t: (bc, t, 0, 0)),
        pl.BlockSpec((1, BB, N), lambda bc, t: (t, bc, 0)),
        pl.BlockSpec((T, BB, N), lambda bc, t: (0, bc, 0)),
        pl.BlockSpec((1, BB, N), lambda bc, t: (t, bc, 0)),
        pl.BlockSpec((BB, 1), lambda bc, t: (bc, 0)),
        pl.BlockSpec((N, N), lambda bc, t: (0, 0)),
        pl.BlockSpec(vp.shape, lambda bc, t: (0, 0)),
        pl.BlockSpec(Wg1.shape, lambda bc, t: (0, 0)),
        pl.BlockSpec(bg1.shape, lambda bc, t: (0, 0)),
        pl.BlockSpec(Wg2.shape, lambda bc, t: (0, 0)),
        pl.BlockSpec(bg2.shape, lambda bc, t: (0, 0)),
    ]
    for w in (Wru2, bru2, Wc2, bc2):
        specs.append(
            pl.BlockSpec(w.shape, lambda bc, t, nd=w.ndim: (0,) * nd))
    return pl.pallas_call(
        _step_kernel,
        grid=grid,
        in_specs=specs,
        out_specs=pl.BlockSpec((BB, N, D), lambda bc, t: (bc, 0, 0)),
        out_shape=jax.ShapeDtypeStruct((B, N, D), jnp.float32),
        scratch_shapes=[
            pltpu.VMEM((BB, N, D), jnp.bfloat16),
            pltpu.VMEM((N, N), jnp.bfloat16),
            pltpu.VMEM((N, N), jnp.bfloat16),
            pltpu.VMEM((BB, N), jnp.float32),
        ],
        compiler_params=pltpu.CompilerParams(
            dimension_semantics=("arbitrary", "arbitrary"),
        ),
    )(obs_emb, maskt, maskt, avgt, lengths, rarity_W, vp, Wg1, bg1, Wg2,
      bg2, Wru2, bru2, Wc2, bc2)


def kernel(obs_emb, adj, observed_mask, observed_tp, tp_emb_tensor, lengths,
           avg_interval, var_prior_emb_tensor, rarity_W, Wg1, bg1, Wg2, bg2,
           Wu, bu, Wr, br, Wc, bc):
    del adj, observed_tp, tp_emb_tensor
    maskt = observed_mask.astype(jnp.float32).transpose(1, 0, 2)
    avgt = avg_interval.transpose(1, 0, 2)

    def repack(w):
        return jnp.concatenate(
            [w[:, :D, :], w[:, D + 1:, :], w[:, D:D + 1, :]], axis=1)

    Wru2 = jnp.concatenate([repack(Wr), repack(Wu)],
                           axis=2).astype(jnp.bfloat16)
    bru2 = jnp.concatenate([br, bu], axis=1)
    return _run(obs_emb, maskt, avgt, lengths, rarity_W,
                var_prior_emb_tensor, Wg1, bg1.reshape(1, -1), Wg2,
                bg2.reshape(1, -1), Wru2, bru2,
                repack(Wc).astype(jnp.bfloat16), bc)


# speedup vs baseline: 2.0591x; 1.1990x over previous
import functools

import jax
import jax.numpy as jnp
from jax.experimental import pallas as pl
from jax.experimental.pallas import tpu as pltpu

B, T, N, D = 128, 24, 100, 64
RARITY_ALPHA = 0.5
BB = 128
NB = B // BB


def _step_kernel(obs_ref, maskt_ref, maskfull_ref, avgt_ref, len_ref,
                 rarw_ref, vp_ref, wg1_ref, bg1_ref, wg2_ref, bg2_ref,
                 wru_ref, bru_ref, wc_ref, bc_ref,
                 out_ref, h_ref, off_ref, roff_ref, vt_ref, eyeb_ref):
    t = pl.program_id(1)

    @pl.when(t == 0)
    def _init():
        pg = jnp.maximum(
            jax.lax.dot(vp_ref[...], wg1_ref[...],
                        preferred_element_type=jnp.float32) + bg1_ref[...],
            0.0)
        pg = jax.lax.dot(pg, wg2_ref[...],
                         preferred_element_type=jnp.float32) + bg2_ref[...]
        nrm = jnp.sqrt(jnp.sum(pg * pg, axis=-1, keepdims=True))
        vn = pg / jnp.maximum(nrm, 1e-12)
        logits = jax.lax.dot_general(
            vn, vn, (((1,), (1,)), ((), ())),
            preferred_element_type=jnp.float32)
        mx = jnp.max(logits, axis=-1, keepdims=True)
        e = jnp.exp(logits - mx)
        adj = e / jnp.sum(e, axis=-1, keepdims=True)
        row_i = jax.lax.broadcasted_iota(jnp.int32, (N, N), 0)
        col_i = jax.lax.broadcasted_iota(jnp.int32, (N, N), 1)
        noteye = (row_i != col_i).astype(jnp.float32)
        off = adj * noteye
        off_ref[...] = off.astype(jnp.bfloat16)
        roff_ref[...] = (off * rarw_ref[...]).astype(jnp.bfloat16)
        vt_ref[...] = jnp.sum(maskfull_ref[...], axis=0)
        h_ref[...] = jnp.zeros_like(h_ref)
        out_ref[...] = jnp.zeros_like(out_ref)
        bi = jax.lax.broadcasted_iota(jnp.int32, (BB, BB), 0)
        bj = jax.lax.broadcasted_iota(jnp.int32, (BB, BB), 1)
        eyeb_ref[...] = (bi == bj).astype(jnp.bfloat16)

    mask = maskt_ref[0].astype(jnp.bfloat16)
    avg = avgt_ref[0]
    vt = vt_ref[...]
    h = h_ref[...]
    cur_obs = jax.lax.dot_general(
        eyeb_ref[...], obs_ref[0].astype(jnp.bfloat16),
        (((1,), (2,)), ((), ())),
        preferred_element_type=jnp.float32
    ).astype(jnp.bfloat16)

    rs = (RARITY_ALPHA * jnp.tanh(avg / (vt + 1.0))).astype(jnp.bfloat16)
    diff = jnp.abs(rs[:, :, None] - rs[:, None, :])
    madj = mask[:, :, None] * mask[:, None, :]
    row_i = jax.lax.broadcasted_iota(jnp.int32, (N, N), 0)
    col_i = jax.lax.broadcasted_iota(jnp.int32, (N, N), 1)
    eyeb = (row_i == col_i).astype(jnp.bfloat16)
    cur_adj = (off_ref[...][None] - roff_ref[...][None] * diff) * madj \
        + eyeb[None]

    xh = jnp.concatenate([cur_obs, h, rs[:, :, None]], axis=-1)
    comb = jax.lax.dot_general(
        cur_adj, xh, (((2,), (1,)), ((0,), (0,))),
        preferred_element_type=jnp.float32
    ).astype(jnp.bfloat16)

    def gate(x, w_ref, b_ref):
        pre = jax.lax.dot_general(
            x, w_ref[...], (((2,), (1,)), ((1,), (0,))),
            preferred_element_type=jnp.float32)
        pre = jnp.transpose(pre, (1, 0, 2))
        return pre + b_ref[...][None]

    ru = jax.nn.sigmoid(gate(comb, wru_ref, bru_ref)).astype(jnp.bfloat16)
    r = ru[:, :, :D]
    u = ru[:, :, D:]

    m = mask[:, :, None]
    h_reset = h * (1.0 + m * (r - 1.0))
    xh_new = jnp.concatenate([cur_obs, h_reset, rs[:, :, None]], axis=-1)
    cand = jnp.tanh(gate(xh_new, wc_ref, bc_ref)).astype(jnp.bfloat16)
    mu = m * u
    h_next = h_reset * (1.0 - mu) + mu * cand
    h_ref[...] = h_next

    end = (len_ref[...] - 1 == t)
    out_ref[...] = jnp.where(end[:, :, None],
                             h_next.astype(jnp.float32), out_ref[...])


@jax.jit
def _run(obs_emb, maskt, avgt, lengths, rarity_W, vp, Wg1, bg1, Wg2, bg2,
         Wru2, bru2, Wc2, bc2):
    grid = (NB, T)
    specs = [
        pl.BlockSpec((1, N, D, BB), lambda bc, t: (t, 0, 0, bc)),
        pl.BlockSpec((1, BB, N), lambda bc, t: (t, bc, 0)),
        pl.BlockSpec((T, BB, N), lambda bc, t: (0, bc, 0)),
        pl.BlockSpec((1, BB, N), lambda bc, t: (t, bc, 0)),
        pl.BlockSpec((BB, 1), lambda bc, t: (bc, 0)),
        pl.BlockSpec((N, N), lambda bc, t: (0, 0)),
        pl.BlockSpec(vp.shape, lambda bc, t: (0, 0)),
        pl.BlockSpec(Wg1.shape, lambda bc, t: (0, 0)),
        pl.BlockSpec(bg1.shape, lambda bc, t: (0, 0)),
        pl.BlockSpec(Wg2.shape, lambda bc, t: (0, 0)),
        pl.BlockSpec(bg2.shape, lambda bc, t: (0, 0)),
    ]
    for w in (Wru2, bru2, Wc2, bc2):
        specs.append(
            pl.BlockSpec(w.shape, lambda bc, t, nd=w.ndim: (0,) * nd))
    return pl.pallas_call(
        _step_kernel,
        grid=grid,
        in_specs=specs,
        out_specs=pl.BlockSpec((BB, N, D), lambda bc, t: (bc, 0, 0)),
        out_shape=jax.ShapeDtypeStruct((B, N, D), jnp.float32),
        scratch_shapes=[
            pltpu.VMEM((BB, N, D), jnp.bfloat16),
            pltpu.VMEM((N, N), jnp.bfloat16),
            pltpu.VMEM((N, N), jnp.bfloat16),
            pltpu.VMEM((BB, N), jnp.float32),
            pltpu.VMEM((BB, BB), jnp.bfloat16),
        ],
        compiler_params=pltpu.CompilerParams(
            dimension_semantics=("arbitrary", "arbitrary"),
        ),
    )(obs_emb, maskt, maskt, avgt, lengths, rarity_W, vp, Wg1, bg1, Wg2,
      bg2, Wru2, bru2, Wc2, bc2)


def kernel(obs_emb, adj, observed_mask, observed_tp, tp_emb_tensor, lengths,
           avg_interval, var_prior_emb_tensor, rarity_W, Wg1, bg1, Wg2, bg2,
           Wu, bu, Wr, br, Wc, bc):
    del adj, observed_tp, tp_emb_tensor
    maskt = observed_mask.astype(jnp.float32).transpose(1, 0, 2)
    avgt = avg_interval.transpose(1, 0, 2)
    obs_p = obs_emb.transpose(1, 2, 3, 0)

    def repack(w):
        return jnp.concatenate(
            [w[:, :D, :], w[:, D + 1:, :], w[:, D:D + 1, :]], axis=1)

    Wru2 = jnp.concatenate([repack(Wr), repack(Wu)],
                           axis=2).astype(jnp.bfloat16)
    bru2 = jnp.concatenate([br, bu], axis=1)
    return _run(obs_p, maskt, avgt, lengths, rarity_W,
                var_prior_emb_tensor, Wg1, bg1.reshape(1, -1), Wg2,
                bg2.reshape(1, -1), Wru2, bru2,
                repack(Wc).astype(jnp.bfloat16), bc)
